# baseline (speedup 1.0000x reference)
import functools

import jax
import jax.numpy as jnp
from jax import lax
from jax.experimental import pallas as pl
from jax.experimental.pallas import tpu as pltpu
from jax.experimental.pallas import tpu_sc as plsc


def _selector_sc(B, S, D, dtype):
    mesh = plsc.VectorSubcoreMesh(core_axis_name="c", subcore_axis_name="s")

    @functools.partial(
        pl.kernel,
        mesh=mesh,
        out_type=jax.ShapeDtypeStruct((B, D), dtype),
        scratch_types=[
            pltpu.VMEM((B,), jnp.int32),
            pltpu.VMEM((B, D), dtype),
            pltpu.SemaphoreType.DMA,
        ],
    )
    def gather_kernel(x_hbm, gidx_hbm, out_hbm, idx_v, rows_v, sem):
        wid = lax.axis_index("s") * 2 + lax.axis_index("c")

        @pl.when(wid == 0)
        def _():
            pltpu.sync_copy(gidx_hbm, idx_v)
            pltpu.async_copy(x_hbm.at[idx_v], rows_v, sem).wait()
            pltpu.sync_copy(rows_v, out_hbm)

    return gather_kernel


def kernel(x, idx):
    B, S, D = x.shape
    x_flat = x.reshape(B * S, D)
    gidx = idx.astype(jnp.int32) + jnp.arange(B, dtype=jnp.int32) * S
    return _selector_sc(B, S, D, x.dtype)(x_flat, gidx)


# speedup vs baseline: 1.0632x; 1.0632x over previous
import functools

import jax
import jax.numpy as jnp
from jax import lax
from jax.experimental import pallas as pl
from jax.experimental.pallas import tpu as pltpu
from jax.experimental.pallas import tpu_sc as plsc


def _selector_sc(B, S, D, dtype):
    mesh = plsc.VectorSubcoreMesh(
        core_axis_name="c", subcore_axis_name="s", num_cores=1
    )

    @functools.partial(
        pl.kernel,
        mesh=mesh,
        out_type=jax.ShapeDtypeStruct((B, D), dtype),
        scratch_types=[
            pltpu.VMEM((B,), jnp.int32),
            pltpu.VMEM((B, D), dtype),
            pltpu.SemaphoreType.DMA,
        ],
    )
    def gather_kernel(x_hbm, gidx_hbm, out_hbm, idx_v, rows_v, sem):
        wid = lax.axis_index("s") * 2 + lax.axis_index("c")

        @pl.when(wid == 0)
        def _():
            pltpu.sync_copy(gidx_hbm, idx_v)
            pltpu.async_copy(x_hbm.at[idx_v], rows_v, sem).wait()
            pltpu.sync_copy(rows_v, out_hbm)

    return gather_kernel


def kernel(x, idx):
    B, S, D = x.shape
    x_flat = x.reshape(B * S, D)
    gidx = idx.astype(jnp.int32) + jnp.arange(B, dtype=jnp.int32) * S
    return _selector_sc(B, S, D, x.dtype)(x_flat, gidx)


# speedup vs baseline: 1.0801x; 1.0159x over previous
import functools

import jax
import jax.numpy as jnp
from jax import lax
from jax.experimental import pallas as pl
from jax.experimental.pallas import tpu as pltpu
from jax.experimental.pallas import tpu_sc as plsc


def _selector_sc(B, S, D, dtype):
    mesh = plsc.VectorSubcoreMesh(
        core_axis_name="c", subcore_axis_name="s", num_cores=1, num_subcores=1
    )

    @functools.partial(
        pl.kernel,
        mesh=mesh,
        out_type=jax.ShapeDtypeStruct((B, D), dtype),
        scratch_types=[
            pltpu.VMEM((B,), jnp.int32),
            pltpu.VMEM((B, D), dtype),
            pltpu.SemaphoreType.DMA,
        ],
    )
    def gather_kernel(x_hbm, gidx_hbm, out_hbm, idx_v, rows_v, sem):
        pltpu.sync_copy(gidx_hbm, idx_v)
        pltpu.async_copy(x_hbm.at[idx_v], rows_v, sem).wait()
        pltpu.sync_copy(rows_v, out_hbm)

    return gather_kernel


def kernel(x, idx):
    B, S, D = x.shape
    x_flat = x.reshape(B * S, D)
    gidx = idx.astype(jnp.int32) + jnp.arange(B, dtype=jnp.int32) * S
    return _selector_sc(B, S, D, x.dtype)(x_flat, gidx)
